# baseline (speedup 1.0000x reference)
import jax
import jax.numpy as jnp
from jax.experimental import pallas as pl
from jax.experimental.pallas import tpu as pltpu


def _fused_kernel(h2, w2, x_ref, wt_ref, b_ref, o_ref, a_scr, t_scr):
    c = x_ref.shape[1]
    a_scr[...] = (x_ref[0, :, pl.ds(0, h2, 2), :]
                  + x_ref[0, :, pl.ds(1, h2, 2), :])
    for i in range(h2):
        t_scr[i] = jnp.transpose(a_scr[:, i, :])
    p3 = t_scr[:, pl.ds(0, w2, 2), :] + t_scr[:, pl.ds(1, w2, 2), :]
    p = p3.reshape(h2 * w2, c)
    y = jnp.dot(p, wt_ref[...], preferred_element_type=jnp.float32)
    y = y + b_ref[...]
    y3 = y.reshape(h2, w2, y.shape[-1])
    for i in range(h2):
        o_ref[0, :, i, :] = jnp.transpose(y3[i])


def kernel(x_nchw, expand_w, expand_b):
    B, C, H, W = x_nchw.shape
    C_out = expand_w.shape[0]
    H2, W2 = H // 2, W // 2
    if (H % 2) or (W % 2):
        x_nchw = x_nchw[:, :, : 2 * H2, : 2 * W2]
        H, W = 2 * H2, 2 * W2

    wt = (jnp.transpose(expand_w) * 0.25).astype(x_nchw.dtype)
    b2 = jnp.asarray(expand_b, jnp.float32).reshape(1, C_out)

    return pl.pallas_call(
        lambda x_ref, wt_ref, b_ref, o_ref, a_scr, t_scr: _fused_kernel(
            H2, W2, x_ref, wt_ref, b_ref, o_ref, a_scr, t_scr),
        out_shape=jax.ShapeDtypeStruct((B, C_out, H2, W2), x_nchw.dtype),
        grid=(B,),
        in_specs=[
            pl.BlockSpec((1, C, H, W), lambda i: (i, 0, 0, 0)),
            pl.BlockSpec((C, C_out), lambda i: (0, 0)),
            pl.BlockSpec((1, C_out), lambda i: (0, 0)),
        ],
        out_specs=pl.BlockSpec((1, C_out, H2, W2), lambda i: (i, 0, 0, 0)),
        scratch_shapes=[
            pltpu.VMEM((C, H2, W), jnp.float32),
            pltpu.VMEM((H2, W, C), jnp.float32),
        ],
        compiler_params=pltpu.CompilerParams(
            dimension_semantics=("parallel",),
            vmem_limit_bytes=64 * 1024 * 1024,
        ),
    )(x_nchw, wt, b2)


# speedup vs baseline: 2.7566x; 2.7566x over previous
import jax
import jax.numpy as jnp
from jax.experimental import pallas as pl
from jax.experimental.pallas import tpu as pltpu


def _fused_kernel(nb, h2, w2, x_ref, wt_ref, b_ref, o_ref, t_scr):
    c = x_ref.shape[1]
    for n in range(nb):
        t = jnp.transpose(x_ref[n])
        t_scr[n] = t.reshape(2 * h2, 2 * w2, c)
    ev, od = pl.ds(0, h2, 2), pl.ds(1, h2, 2)
    evw, odw = pl.ds(0, w2, 2), pl.ds(1, w2, 2)
    for n in range(nb):
        p3 = (t_scr[n, ev, evw, :] + t_scr[n, ev, odw, :]
              + t_scr[n, od, evw, :] + t_scr[n, od, odw, :])
        p = p3.reshape(h2 * w2, c)
        y = jnp.dot(p, wt_ref[...], preferred_element_type=jnp.float32)
        y = y + b_ref[...]
        o_ref[n] = jnp.transpose(y)


def kernel(x_nchw, expand_w, expand_b):
    B, C, H, W = x_nchw.shape
    C_out = expand_w.shape[0]
    H2, W2 = H // 2, W // 2
    if (H % 2) or (W % 2):
        x_nchw = x_nchw[:, :, : 2 * H2, : 2 * W2]
        H, W = 2 * H2, 2 * W2

    NB = 2 if B % 2 == 0 else 1
    xf = x_nchw.reshape(B, C, H * W)
    wt = (jnp.transpose(expand_w) * 0.25).astype(x_nchw.dtype)
    b2 = jnp.asarray(expand_b, jnp.float32).reshape(1, C_out)

    out_flat = pl.pallas_call(
        lambda x_ref, wt_ref, b_ref, o_ref, t_scr: _fused_kernel(
            NB, H2, W2, x_ref, wt_ref, b_ref, o_ref, t_scr),
        out_shape=jax.ShapeDtypeStruct((B, C_out, H2 * W2), x_nchw.dtype),
        grid=(B // NB,),
        in_specs=[
            pl.BlockSpec((NB, C, H * W), lambda i: (i, 0, 0)),
            pl.BlockSpec((C, C_out), lambda i: (0, 0)),
            pl.BlockSpec((1, C_out), lambda i: (0, 0)),
        ],
        out_specs=pl.BlockSpec((NB, C_out, H2 * W2), lambda i: (i, 0, 0)),
        scratch_shapes=[pltpu.VMEM((NB, H, W, C), jnp.float32)],
        compiler_params=pltpu.CompilerParams(
            dimension_semantics=("parallel",),
            vmem_limit_bytes=64 * 1024 * 1024,
        ),
    )(xf, wt, b2)

    return out_flat.reshape(B, C_out, H2, W2)
